# Initial kernel scaffold; baseline (speedup 1.0000x reference)
import jax
import jax.numpy as jnp
from jax.experimental import pallas as pl


def kernel(x, rand_indices, rand_signs):
    raise NotImplementedError("write your pallas kernel here")


# speedup vs baseline: 12.2377x; 12.2377x over previous
import functools

import jax
import jax.numpy as jnp
from jax.experimental import pallas as pl
from jax.experimental.pallas import tpu as pltpu

ORIGINAL_DIM = 4096
PROJ_DIM = 1024
C = 4
BATCH = 2048

BM = 512


def _sjlt_kernel(idx_ref, sign_ref, x_ref, o_ref, s_ref):
    @pl.when(pl.program_id(0) == 0)
    def _build_s():
        DB = 512
        p = jax.lax.broadcasted_iota(jnp.int32, (DB, PROJ_DIM), 1)
        for d0 in range(0, ORIGINAL_DIM, DB):
            idx = idx_ref[d0:d0 + DB, :]
            sign = sign_ref[d0:d0 + DB, :]
            acc = jnp.zeros((DB, PROJ_DIM), jnp.float32)
            for j in range(C):
                acc += jnp.where(idx[:, j][:, None] == p,
                                 sign[:, j][:, None], 0.0)
            s_ref[d0:d0 + DB, :] = acc

    o_ref[...] = jnp.dot(x_ref[...], s_ref[...],
                         preferred_element_type=jnp.float32)


@jax.jit
def kernel(x, rand_indices, rand_signs):
    idx = rand_indices.astype(jnp.int32)
    sign = rand_signs.astype(jnp.float32)
    grid = (BATCH // BM,)
    return pl.pallas_call(
        _sjlt_kernel,
        grid=grid,
        in_specs=[
            pl.BlockSpec((ORIGINAL_DIM, C), lambda i: (0, 0)),
            pl.BlockSpec((ORIGINAL_DIM, C), lambda i: (0, 0)),
            pl.BlockSpec((BM, ORIGINAL_DIM), lambda i: (i, 0)),
        ],
        out_specs=pl.BlockSpec((BM, PROJ_DIM), lambda i: (i, 0)),
        out_shape=jax.ShapeDtypeStruct((BATCH, PROJ_DIM), jnp.float32),
        scratch_shapes=[pltpu.VMEM((ORIGINAL_DIM, PROJ_DIM), jnp.float32)],
    )(idx, sign, x)
